# Initial kernel scaffold; baseline (speedup 1.0000x reference)
import jax
import jax.numpy as jnp
from jax.experimental import pallas as pl


def kernel(inputs, codebook):
    raise NotImplementedError("write your pallas kernel here")


# speedup vs baseline: 1.8734x; 1.8734x over previous
import functools

import jax
import jax.numpy as jnp
from jax import lax
from jax.experimental import pallas as pl
from jax.experimental.pallas import tpu as pltpu
from jax.experimental.pallas import tpu_sc as plsc

_D = 10
_DP = 16
_NC, _NS, _L = 2, 16, 16
_NW = _NC * _NS
_N = 8 * 4096
_TPW = _N // _NW
_GROUPS = _TPW // _L
_CHUNK = _TPW * _D
_BLK = 4096
_GRID = _N // _BLK

def _sc_mesh():
    return plsc.VectorSubcoreMesh(
        core_axis_name="c", subcore_axis_name="s",
        num_cores=_NC, num_subcores=_NS)


_SC_PARAMS = pltpu.CompilerParams(needs_layout_passes=False)


def _worker_id():
    return lax.axis_index("s") * _NC + lax.axis_index("c")


def _sc_transpose_body(x_hbm, xT_hbm, x_v, xT_v):
    wid = _worker_id()
    pltpu.sync_copy(x_hbm.at[pl.ds(wid * _CHUNK, _CHUNK)], x_v)
    lane = lax.iota(jnp.int32, _L)

    def group(g, carry):
        pos0 = (g * _L + lane) * _D
        for d in range(_D):
            xT_v[d, pl.ds(g * _L, _L)] = plsc.load_gather(x_v, [pos0 + d])
        return carry

    lax.fori_loop(0, _GROUPS, group, jnp.int32(0))
    for d in range(_D):
        pltpu.sync_copy(xT_v.at[d], xT_hbm.at[d, pl.ds(wid * _TPW, _TPW)])


@functools.lru_cache(maxsize=None)
def _sc_transpose():
    return pl.kernel(
        _sc_transpose_body,
        out_type=jax.ShapeDtypeStruct((_DP, _N), jnp.float32),
        mesh=_sc_mesh(),
        scratch_types=[
            pltpu.VMEM((_CHUNK,), jnp.float32),
            pltpu.VMEM((_D, _TPW), jnp.float32),
        ],
        compiler_params=_SC_PARAMS,
    )


def _tc_math_body(xT_ref, bitsT_ref, idx_ref, loss_ref):
    step = pl.program_id(0)

    xs = [xT_ref[d:d + 1, :] for d in range(_D)]
    xbs = [x.astype(jnp.bfloat16).astype(jnp.float32) for x in xs]
    bits = [xb > 0.5 for xb in xbs]
    ties = [xb == 0.5 for xb in xbs]

    x2 = xs[0] * xs[0]
    for d in range(1, _D):
        x2 = x2 + xs[d] * xs[d]

    big = jnp.int32(99)
    d1 = jnp.full_like(x2, big, dtype=jnp.int32)
    for d in range(_D - 1, -1, -1):
        d1 = jnp.where(ties[d], jnp.int32(d), d1)
    d2 = jnp.full_like(x2, big, dtype=jnp.int32)
    for d in range(_D - 1, -1, -1):
        d2 = jnp.where(ties[d] & (jnp.int32(d) > d1), jnp.int32(d), d2)
    has1 = d1 < big
    has2 = d2 < big

    zero = jnp.zeros_like(x2)
    dots = [zero, zero, zero, zero]
    c2 = zero
    for d in range(_D):
        v = jnp.where(bits[d], xbs[d], 0.0)
        c2 = c2 + jnp.where(bits[d], 1.0, 0.0)
        is1 = d1 == d
        is2 = d2 == d
        vh = jnp.where(is1, jnp.float32(0.5), v)
        vl = jnp.where(is2, jnp.float32(0.5), v)
        vhl = jnp.where(is1 | is2, jnp.float32(0.5), v)
        dots = [dots[0] + v, dots[1] + vl, dots[2] + vh, dots[3] + vhl]

    f1 = jnp.where(has1, 1.0, 0.0)
    f2 = jnp.where(has2, 1.0, 0.0)
    c2s = [c2, c2 + f2, c2 + f1, c2 + f1 + f2]
    valid = [None,
             has2,
             has1,
             has1 & has2]
    inf = jnp.float32(jnp.inf)

    def dist(c2_c, dot_c):
        d2v = (x2 + c2_c) - 2.0 * dot_c
        return lax.sqrt(jnp.maximum(d2v, 0.0))

    best = dist(c2s[0], dots[0])
    bhi_i = jnp.zeros_like(d1)
    blo_i = jnp.zeros_like(d1)
    for c in (1, 2, 3):
        dc = jnp.where(valid[c], dist(c2s[c], dots[c]), inf)
        win = dc < best
        best = jnp.where(win, dc, best)
        bhi_i = jnp.where(win, jnp.int32(c >> 1), bhi_i)
        blo_i = jnp.where(win, jnp.int32(c & 1), blo_i)
    bhi = bhi_i == 1
    blo = blo_i == 1

    idx = jnp.zeros_like(d1)
    part = jnp.zeros_like(x2)
    for d in range(_D):
        fb = bits[d] | (bhi & (d1 == d)) | (blo & (d2 == d))
        q = jnp.where(fb, jnp.float32(1.0), jnp.float32(0.0))
        bitsT_ref[d:d + 1, :] = q
        idx = idx + jnp.where(fb, jnp.int32(1 << (_D - 1 - d)), jnp.int32(0))
        diff = q - xs[d]
        part = part + diff * diff
    idx_ref[...] = idx[None]

    @pl.when(step == 0)
    def _():
        loss_ref[...] = jnp.zeros_like(loss_ref)

    loss_ref[...] += jnp.sum(part)[None, None] * jnp.float32(1.0 / (_N * _D))


_tc_math = pl.pallas_call(
    _tc_math_body,
    grid=(_GRID,),
    in_specs=[pl.BlockSpec((_DP, _BLK), lambda i: (0, i))],
    out_specs=[
        pl.BlockSpec((_DP, _BLK), lambda i: (0, i)),
        pl.BlockSpec((1, 1, _BLK), lambda i: (i, 0, 0)),
        pl.BlockSpec((1, 1), lambda i: (0, 0)),
    ],
    out_shape=[
        jax.ShapeDtypeStruct((_DP, _N), jnp.float32),
        jax.ShapeDtypeStruct((_GRID, 1, _BLK), jnp.int32),
        jax.ShapeDtypeStruct((1, 1), jnp.float32),
    ],
)


def _sc_untranspose_body(bT_hbm, q_hbm, bT_v, q_v):
    wid = _worker_id()
    for d in range(_D):
        pltpu.sync_copy(bT_hbm.at[d, pl.ds(wid * _TPW, _TPW)], bT_v.at[d])
    lane = lax.iota(jnp.int32, _L)

    def group(g, carry):
        pos0 = (g * _L + lane) * _D
        for d in range(_D):
            plsc.store_scatter(q_v, [pos0 + d], bT_v[d, pl.ds(g * _L, _L)])
        return carry

    lax.fori_loop(0, _GROUPS, group, jnp.int32(0))
    pltpu.sync_copy(q_v, q_hbm.at[pl.ds(wid * _CHUNK, _CHUNK)])


@functools.lru_cache(maxsize=None)
def _sc_untranspose():
    return pl.kernel(
        _sc_untranspose_body,
        out_type=jax.ShapeDtypeStruct((_N * _D,), jnp.float32),
        mesh=_sc_mesh(),
        scratch_types=[
            pltpu.VMEM((_D, _TPW), jnp.float32),
            pltpu.VMEM((_CHUNK,), jnp.float32),
        ],
        compiler_params=_SC_PARAMS,
    )


def kernel(inputs, codebook):
    del codebook
    flat = inputs.reshape(-1)
    xT = _sc_transpose()(flat)
    bitsT, idx, loss = _tc_math(xT)
    q_flat = _sc_untranspose()(bitsT)
    return (q_flat.reshape(inputs.shape), loss.reshape(()), idx.reshape(_N))


# speedup vs baseline: 2.1719x; 1.1594x over previous
import functools

import jax
import jax.numpy as jnp
from jax import lax
from jax.experimental import pallas as pl
from jax.experimental.pallas import tpu as pltpu
from jax.experimental.pallas import tpu_sc as plsc

_D = 10
_DP = 16
_NC, _NS, _L = 2, 16, 16
_NW = _NC * _NS
_N = 8 * 4096
_TPW = _N // _NW
_GROUPS = _TPW // _L
_CHUNK = _TPW * _D
_ROWS = 8
_COLS = _N // _ROWS
_WPR = _COLS // _TPW
_BLK = 1024
_GRID = _COLS // _BLK

_SC_PARAMS = pltpu.CompilerParams(needs_layout_passes=False)


def _sc_mesh():
    return plsc.VectorSubcoreMesh(
        core_axis_name="c", subcore_axis_name="s",
        num_cores=_NC, num_subcores=_NS)


def _worker_id():
    return lax.axis_index("s") * _NC + lax.axis_index("c")


def _sc_transpose_body(x_hbm, xT_hbm, x_v, xT_v):
    wid = _worker_id()
    pltpu.sync_copy(x_hbm.at[pl.ds(wid * _CHUNK, _CHUNK)], x_v)
    lane = lax.iota(jnp.int32, _L)

    def group(g, carry):
        pos0 = (g * _L + lane) * _D
        for d in range(_D):
            xT_v[d, pl.ds(g * _L, _L)] = plsc.load_gather(x_v, [pos0 + d])
        return carry

    lax.fori_loop(0, _GROUPS, group, jnp.int32(0))
    row = wid // _WPR
    col0 = (wid % _WPR) * _TPW
    for d in range(_D):
        pltpu.sync_copy(xT_v.at[d], xT_hbm.at[d, row, pl.ds(col0, _TPW)])


@functools.lru_cache(maxsize=None)
def _sc_transpose():
    return pl.kernel(
        _sc_transpose_body,
        out_type=jax.ShapeDtypeStruct((_DP, _ROWS, _COLS), jnp.float32),
        mesh=_sc_mesh(),
        scratch_types=[
            pltpu.VMEM((_CHUNK,), jnp.float32),
            pltpu.VMEM((_D, _TPW), jnp.float32),
        ],
        compiler_params=_SC_PARAMS,
    )


def _tc_math_body(xT_ref, bitsT_ref, idx_ref, loss_ref):
    step = pl.program_id(0)

    xs = [xT_ref[d] for d in range(_D)]
    xbs = [x.astype(jnp.bfloat16).astype(jnp.float32) for x in xs]
    bits = [xb > 0.5 for xb in xbs]
    ties = [xb == 0.5 for xb in xbs]

    x2 = xs[0] * xs[0]
    for d in range(1, _D):
        x2 = x2 + xs[d] * xs[d]

    big = jnp.int32(99)
    d1 = jnp.full_like(x2, big, dtype=jnp.int32)
    for d in range(_D - 1, -1, -1):
        d1 = jnp.where(ties[d], jnp.int32(d), d1)
    d2 = jnp.full_like(x2, big, dtype=jnp.int32)
    for d in range(_D - 1, -1, -1):
        d2 = jnp.where(ties[d] & (jnp.int32(d) > d1), jnp.int32(d), d2)
    has1 = d1 < big
    has2 = d2 < big

    zero = jnp.zeros_like(x2)
    dots = [zero, zero, zero, zero]
    c2 = zero
    for d in range(_D):
        v = jnp.where(bits[d], xbs[d], 0.0)
        c2 = c2 + jnp.where(bits[d], 1.0, 0.0)
        is1 = d1 == d
        is2 = d2 == d
        vh = jnp.where(is1, jnp.float32(0.5), v)
        vl = jnp.where(is2, jnp.float32(0.5), v)
        vhl = jnp.where(is1 | is2, jnp.float32(0.5), v)
        dots = [dots[0] + v, dots[1] + vl, dots[2] + vh, dots[3] + vhl]

    f1 = jnp.where(has1, 1.0, 0.0)
    f2 = jnp.where(has2, 1.0, 0.0)
    c2s = [c2, c2 + f2, c2 + f1, c2 + f1 + f2]
    valid = [None,
             has2,
             has1,
             has1 & has2]
    inf = jnp.float32(jnp.inf)

    def dist(c2_c, dot_c):
        d2v = (x2 + c2_c) - 2.0 * dot_c
        return lax.sqrt(jnp.maximum(d2v, 0.0))

    best = dist(c2s[0], dots[0])
    bhi_i = jnp.zeros_like(d1)
    blo_i = jnp.zeros_like(d1)
    for c in (1, 2, 3):
        dc = jnp.where(valid[c], dist(c2s[c], dots[c]), inf)
        win = dc < best
        best = jnp.where(win, dc, best)
        bhi_i = jnp.where(win, jnp.int32(c >> 1), bhi_i)
        blo_i = jnp.where(win, jnp.int32(c & 1), blo_i)
    bhi = bhi_i == 1
    blo = blo_i == 1

    idx = jnp.zeros_like(d1)
    part = jnp.zeros_like(x2)
    for d in range(_D):
        fb = bits[d] | (bhi & (d1 == d)) | (blo & (d2 == d))
        q = jnp.where(fb, jnp.float32(1.0), jnp.float32(0.0))
        bitsT_ref[d] = q
        idx = idx + jnp.where(fb, jnp.int32(1 << (_D - 1 - d)), jnp.int32(0))
        diff = q - xs[d]
        part = part + diff * diff
    idx_ref[...] = idx

    @pl.when(step == 0)
    def _():
        loss_ref[...] = jnp.zeros_like(loss_ref)

    loss_ref[...] += jnp.sum(part)[None, None] * jnp.float32(1.0 / (_N * _D))


_tc_math = pl.pallas_call(
    _tc_math_body,
    grid=(_GRID,),
    in_specs=[pl.BlockSpec((_DP, _ROWS, _BLK), lambda i: (0, 0, i))],
    out_specs=[
        pl.BlockSpec((_DP, _ROWS, _BLK), lambda i: (0, 0, i)),
        pl.BlockSpec((_ROWS, _BLK), lambda i: (0, i)),
        pl.BlockSpec((1, 1), lambda i: (0, 0)),
    ],
    out_shape=[
        jax.ShapeDtypeStruct((_DP, _ROWS, _COLS), jnp.float32),
        jax.ShapeDtypeStruct((_ROWS, _COLS), jnp.int32),
        jax.ShapeDtypeStruct((1, 1), jnp.float32),
    ],
)


def _sc_untranspose_body(bT_hbm, q_hbm, bT_v, q_v):
    wid = _worker_id()
    row = wid // _WPR
    col0 = (wid % _WPR) * _TPW
    for d in range(_D):
        pltpu.sync_copy(bT_hbm.at[d, row, pl.ds(col0, _TPW)], bT_v.at[d])
    lane = lax.iota(jnp.int32, _L)

    def group(g, carry):
        pos0 = (g * _L + lane) * _D
        for d in range(_D):
            plsc.store_scatter(q_v, [pos0 + d], bT_v[d, pl.ds(g * _L, _L)])
        return carry

    lax.fori_loop(0, _GROUPS, group, jnp.int32(0))
    pltpu.sync_copy(q_v, q_hbm.at[pl.ds(wid * _CHUNK, _CHUNK)])


@functools.lru_cache(maxsize=None)
def _sc_untranspose():
    return pl.kernel(
        _sc_untranspose_body,
        out_type=jax.ShapeDtypeStruct((_N * _D,), jnp.float32),
        mesh=_sc_mesh(),
        scratch_types=[
            pltpu.VMEM((_D, _TPW), jnp.float32),
            pltpu.VMEM((_CHUNK,), jnp.float32),
        ],
        compiler_params=_SC_PARAMS,
    )


def kernel(inputs, codebook):
    del codebook
    flat = inputs.reshape(-1)
    xT = _sc_transpose()(flat)
    bitsT, idx, loss = _tc_math(xT)
    q_flat = _sc_untranspose()(bitsT)
    return (q_flat.reshape(inputs.shape), loss.reshape(()), idx.reshape(_N))


# speedup vs baseline: 2.3844x; 1.0978x over previous
import functools

import jax
import jax.numpy as jnp
from jax import lax
from jax.experimental import pallas as pl
from jax.experimental.pallas import tpu as pltpu
from jax.experimental.pallas import tpu_sc as plsc

_D = 10
_NC, _NS, _L = 2, 16, 16
_NW = _NC * _NS
_N = 8 * 4096
_TPW = _N // _NW
_GROUPS = _TPW // _L
_CHUNK = _TPW * _D
_ROWS = 8
_COLS = _N // _ROWS
_WPR = _COLS // _TPW
_BLK = 2048
_GRID = _COLS // _BLK

_SC_PARAMS = pltpu.CompilerParams(needs_layout_passes=False)


def _sc_mesh():
    return plsc.VectorSubcoreMesh(
        core_axis_name="c", subcore_axis_name="s",
        num_cores=_NC, num_subcores=_NS)


def _worker_id():
    return lax.axis_index("s") * _NC + lax.axis_index("c")


def _sc_transpose_body(x_hbm, xT_hbm, x_v, xT_v):
    wid = _worker_id()
    pltpu.sync_copy(x_hbm.at[pl.ds(wid * _CHUNK, _CHUNK)], x_v)
    lane = lax.iota(jnp.int32, _L)

    def group(g, carry):
        pos0 = (g * _L + lane) * _D
        for d in range(_D):
            xT_v[d, pl.ds(g * _L, _L)] = plsc.load_gather(x_v, [pos0 + d])
        return carry

    lax.fori_loop(0, _GROUPS, group, jnp.int32(0))
    row = wid // _WPR
    col0 = (wid % _WPR) * _TPW
    for d in range(_D):
        pltpu.sync_copy(xT_v.at[d], xT_hbm.at[d, row, pl.ds(col0, _TPW)])


@functools.lru_cache(maxsize=None)
def _sc_transpose():
    return pl.kernel(
        _sc_transpose_body,
        out_type=jax.ShapeDtypeStruct((_D, _ROWS, _COLS), jnp.float32),
        mesh=_sc_mesh(),
        scratch_types=[
            pltpu.VMEM((_CHUNK,), jnp.float32),
            pltpu.VMEM((_D, _TPW), jnp.float32),
        ],
        compiler_params=_SC_PARAMS,
    )


def _tc_math_body(xT_ref, idx_ref, loss_ref):
    step = pl.program_id(0)

    xs = [xT_ref[d] for d in range(_D)]
    xbs = [x.astype(jnp.bfloat16).astype(jnp.float32) for x in xs]
    bits = [xb > 0.5 for xb in xbs]
    ties = [xb == 0.5 for xb in xbs]

    x2 = xs[0] * xs[0]
    for d in range(1, _D):
        x2 = x2 + xs[d] * xs[d]

    big = jnp.int32(99)
    d1 = jnp.full_like(x2, big, dtype=jnp.int32)
    for d in range(_D - 1, -1, -1):
        d1 = jnp.where(ties[d], jnp.int32(d), d1)
    d2 = jnp.full_like(x2, big, dtype=jnp.int32)
    for d in range(_D - 1, -1, -1):
        d2 = jnp.where(ties[d] & (jnp.int32(d) > d1), jnp.int32(d), d2)
    has1 = d1 < big
    has2 = d2 < big

    zero = jnp.zeros_like(x2)
    dots = [zero, zero, zero, zero]
    c2 = zero
    for d in range(_D):
        v = jnp.where(bits[d], xbs[d], 0.0)
        c2 = c2 + jnp.where(bits[d], 1.0, 0.0)
        is1 = d1 == d
        is2 = d2 == d
        vh = jnp.where(is1, jnp.float32(0.5), v)
        vl = jnp.where(is2, jnp.float32(0.5), v)
        vhl = jnp.where(is1 | is2, jnp.float32(0.5), v)
        dots = [dots[0] + v, dots[1] + vl, dots[2] + vh, dots[3] + vhl]

    f1 = jnp.where(has1, 1.0, 0.0)
    f2 = jnp.where(has2, 1.0, 0.0)
    c2s = [c2, c2 + f2, c2 + f1, c2 + f1 + f2]
    valid = [None,
             has2,
             has1,
             has1 & has2]
    inf = jnp.float32(jnp.inf)

    def dist(c2_c, dot_c):
        d2v = (x2 + c2_c) - 2.0 * dot_c
        return lax.sqrt(jnp.maximum(d2v, 0.0))

    best = dist(c2s[0], dots[0])
    bhi_i = jnp.zeros_like(d1)
    blo_i = jnp.zeros_like(d1)
    for c in (1, 2, 3):
        dc = jnp.where(valid[c], dist(c2s[c], dots[c]), inf)
        win = dc < best
        best = jnp.where(win, dc, best)
        bhi_i = jnp.where(win, jnp.int32(c >> 1), bhi_i)
        blo_i = jnp.where(win, jnp.int32(c & 1), blo_i)
    bhi = bhi_i == 1
    blo = blo_i == 1

    idx = jnp.zeros_like(d1)
    part = jnp.zeros_like(x2)
    for d in range(_D):
        fb = bits[d] | (bhi & (d1 == d)) | (blo & (d2 == d))
        q = jnp.where(fb, jnp.float32(1.0), jnp.float32(0.0))
        idx = idx + jnp.where(fb, jnp.int32(1 << (_D - 1 - d)), jnp.int32(0))
        diff = q - xs[d]
        part = part + diff * diff
    idx_ref[...] = idx

    @pl.when(step == 0)
    def _():
        loss_ref[...] = jnp.zeros_like(loss_ref)

    loss_ref[...] += jnp.sum(part)[None, None] * jnp.float32(1.0 / (_N * _D))


_tc_math = pl.pallas_call(
    _tc_math_body,
    grid=(_GRID,),
    in_specs=[pl.BlockSpec((_D, _ROWS, _BLK), lambda i: (0, 0, i))],
    out_specs=[
        pl.BlockSpec((_ROWS, _BLK), lambda i: (0, i)),
        pl.BlockSpec((1, 1), lambda i: (0, 0)),
    ],
    out_shape=[
        jax.ShapeDtypeStruct((_ROWS, _COLS), jnp.int32),
        jax.ShapeDtypeStruct((1, 1), jnp.float32),
    ],
)


def _sc_untranspose_body(idx_hbm, q_hbm, idx_v, q_v):
    wid = _worker_id()
    row = wid // _WPR
    col0 = (wid % _WPR) * _TPW
    pltpu.sync_copy(idx_hbm.at[row, pl.ds(col0, _TPW)], idx_v)
    lane = lax.iota(jnp.int32, _L)

    def group(g, carry):
        pos0 = (g * _L + lane) * _D
        iv = idx_v[pl.ds(g * _L, _L)]
        for d in range(_D):
            bit = lax.shift_right_logical(iv, jnp.int32(_D - 1 - d)) & 1
            plsc.store_scatter(q_v, [pos0 + d], bit.astype(jnp.float32))
        return carry

    lax.fori_loop(0, _GROUPS, group, jnp.int32(0))
    pltpu.sync_copy(q_v, q_hbm.at[pl.ds(wid * _CHUNK, _CHUNK)])


@functools.lru_cache(maxsize=None)
def _sc_untranspose():
    return pl.kernel(
        _sc_untranspose_body,
        out_type=jax.ShapeDtypeStruct((_N * _D,), jnp.float32),
        mesh=_sc_mesh(),
        scratch_types=[
            pltpu.VMEM((_TPW,), jnp.int32),
            pltpu.VMEM((_CHUNK,), jnp.float32),
        ],
        compiler_params=_SC_PARAMS,
    )


def kernel(inputs, codebook):
    del codebook
    flat = inputs.reshape(-1)
    xT = _sc_transpose()(flat)
    idx, loss = _tc_math(xT)
    q_flat = _sc_untranspose()(idx)
    return (q_flat.reshape(inputs.shape), loss.reshape(()), idx.reshape(_N))


# speedup vs baseline: 2.3855x; 1.0004x over previous
import functools

import jax
import jax.numpy as jnp
from jax import lax
from jax.experimental import pallas as pl
from jax.experimental.pallas import tpu as pltpu
from jax.experimental.pallas import tpu_sc as plsc

_D = 10
_NC, _NS, _L = 2, 16, 16
_NW = _NC * _NS
_N = 8 * 4096
_TPW = _N // _NW
_GROUPS = _TPW // _L
_CHUNK = _TPW * _D
_ROWS = 8
_COLS = _N // _ROWS
_WPR = _COLS // _TPW
_BLK = 2048
_GRID = _COLS // _BLK

_SC_PARAMS = pltpu.CompilerParams(needs_layout_passes=False,
                                  skip_device_barrier=True)


def _sc_mesh():
    return plsc.VectorSubcoreMesh(
        core_axis_name="c", subcore_axis_name="s",
        num_cores=_NC, num_subcores=_NS)


def _worker_id():
    return lax.axis_index("s") * _NC + lax.axis_index("c")


def _sc_transpose_body(x_hbm, xT_hbm, x_v, xT_v):
    wid = _worker_id()
    pltpu.sync_copy(x_hbm.at[pl.ds(wid * _CHUNK, _CHUNK)], x_v)
    lane = lax.iota(jnp.int32, _L)

    def group(g, carry):
        pos0 = (g * _L + lane) * _D
        for d in range(_D):
            xT_v[d, pl.ds(g * _L, _L)] = plsc.load_gather(x_v, [pos0 + d])
        return carry

    lax.fori_loop(0, _GROUPS, group, jnp.int32(0))
    row = wid // _WPR
    col0 = (wid % _WPR) * _TPW
    for d in range(_D):
        pltpu.sync_copy(xT_v.at[d], xT_hbm.at[d, row, pl.ds(col0, _TPW)])


@functools.lru_cache(maxsize=None)
def _sc_transpose():
    return pl.kernel(
        _sc_transpose_body,
        out_type=jax.ShapeDtypeStruct((_D, _ROWS, _COLS), jnp.float32),
        mesh=_sc_mesh(),
        scratch_types=[
            pltpu.VMEM((_CHUNK,), jnp.float32),
            pltpu.VMEM((_D, _TPW), jnp.float32),
        ],
        compiler_params=_SC_PARAMS,
    )


def _tc_math_body(xT_ref, idx_ref, loss_ref):
    step = pl.program_id(0)

    xs = [xT_ref[d] for d in range(_D)]
    xbs = [x.astype(jnp.bfloat16).astype(jnp.float32) for x in xs]
    bits = [xb > 0.5 for xb in xbs]
    ties = [xb == 0.5 for xb in xbs]

    x2 = xs[0] * xs[0]
    for d in range(1, _D):
        x2 = x2 + xs[d] * xs[d]

    big = jnp.int32(99)
    d1 = jnp.full_like(x2, big, dtype=jnp.int32)
    for d in range(_D - 1, -1, -1):
        d1 = jnp.where(ties[d], jnp.int32(d), d1)
    d2 = jnp.full_like(x2, big, dtype=jnp.int32)
    for d in range(_D - 1, -1, -1):
        d2 = jnp.where(ties[d] & (jnp.int32(d) > d1), jnp.int32(d), d2)
    has1 = d1 < big
    has2 = d2 < big

    zero = jnp.zeros_like(x2)
    dots = [zero, zero, zero, zero]
    c2 = zero
    for d in range(_D):
        v = jnp.where(bits[d], xbs[d], 0.0)
        c2 = c2 + jnp.where(bits[d], 1.0, 0.0)
        is1 = d1 == d
        is2 = d2 == d
        vh = jnp.where(is1, jnp.float32(0.5), v)
        vl = jnp.where(is2, jnp.float32(0.5), v)
        vhl = jnp.where(is1 | is2, jnp.float32(0.5), v)
        dots = [dots[0] + v, dots[1] + vl, dots[2] + vh, dots[3] + vhl]

    f1 = jnp.where(has1, 1.0, 0.0)
    f2 = jnp.where(has2, 1.0, 0.0)
    c2s = [c2, c2 + f2, c2 + f1, c2 + f1 + f2]
    valid = [None,
             has2,
             has1,
             has1 & has2]
    inf = jnp.float32(jnp.inf)

    def dist(c2_c, dot_c):
        d2v = (x2 + c2_c) - 2.0 * dot_c
        return lax.sqrt(jnp.maximum(d2v, 0.0))

    best = dist(c2s[0], dots[0])
    bhi_i = jnp.zeros_like(d1)
    blo_i = jnp.zeros_like(d1)
    for c in (1, 2, 3):
        dc = jnp.where(valid[c], dist(c2s[c], dots[c]), inf)
        win = dc < best
        best = jnp.where(win, dc, best)
        bhi_i = jnp.where(win, jnp.int32(c >> 1), bhi_i)
        blo_i = jnp.where(win, jnp.int32(c & 1), blo_i)
    bhi = bhi_i == 1
    blo = blo_i == 1

    idx = jnp.zeros_like(d1)
    part = jnp.zeros_like(x2)
    for d in range(_D):
        fb = bits[d] | (bhi & (d1 == d)) | (blo & (d2 == d))
        q = jnp.where(fb, jnp.float32(1.0), jnp.float32(0.0))
        idx = idx + jnp.where(fb, jnp.int32(1 << (_D - 1 - d)), jnp.int32(0))
        diff = q - xs[d]
        part = part + diff * diff
    idx_ref[...] = idx

    @pl.when(step == 0)
    def _():
        loss_ref[...] = jnp.zeros_like(loss_ref)

    loss_ref[...] += jnp.sum(part)[None, None] * jnp.float32(1.0 / (_N * _D))


_tc_math = pl.pallas_call(
    _tc_math_body,
    grid=(_GRID,),
    in_specs=[pl.BlockSpec((_D, _ROWS, _BLK), lambda i: (0, 0, i))],
    out_specs=[
        pl.BlockSpec((_ROWS, _BLK), lambda i: (0, i)),
        pl.BlockSpec((1, 1), lambda i: (0, 0)),
    ],
    out_shape=[
        jax.ShapeDtypeStruct((_ROWS, _COLS), jnp.int32),
        jax.ShapeDtypeStruct((1, 1), jnp.float32),
    ],
)


def _sc_untranspose_body(idx_hbm, q_hbm, idx_v, q_v):
    wid = _worker_id()
    row = wid // _WPR
    col0 = (wid % _WPR) * _TPW
    pltpu.sync_copy(idx_hbm.at[row, pl.ds(col0, _TPW)], idx_v)
    lane = lax.iota(jnp.int32, _L)

    def group(g, carry):
        pos0 = (g * _L + lane) * _D
        iv = idx_v[pl.ds(g * _L, _L)]
        for d in range(_D):
            bit = lax.shift_right_logical(iv, jnp.int32(_D - 1 - d)) & 1
            plsc.store_scatter(q_v, [pos0 + d], bit.astype(jnp.float32))
        return carry

    lax.fori_loop(0, _GROUPS, group, jnp.int32(0))
    pltpu.sync_copy(q_v, q_hbm.at[pl.ds(wid * _CHUNK, _CHUNK)])


@functools.lru_cache(maxsize=None)
def _sc_untranspose():
    return pl.kernel(
        _sc_untranspose_body,
        out_type=jax.ShapeDtypeStruct((_N * _D,), jnp.float32),
        mesh=_sc_mesh(),
        scratch_types=[
            pltpu.VMEM((_TPW,), jnp.int32),
            pltpu.VMEM((_CHUNK,), jnp.float32),
        ],
        compiler_params=_SC_PARAMS,
    )


def kernel(inputs, codebook):
    del codebook
    flat = inputs.reshape(-1)
    xT = _sc_transpose()(flat)
    idx, loss = _tc_math(xT)
    q_flat = _sc_untranspose()(idx)
    return (q_flat.reshape(inputs.shape), loss.reshape(()), idx.reshape(_N))
